# baseline (speedup 1.0000x reference)
import functools

import jax
import jax.numpy as jnp
from jax import lax
from jax.experimental import pallas as pl
from jax.experimental.pallas import tpu as pltpu
from jax.experimental.pallas import tpu_sc as plsc

_B, _N, _C = 4, 2048, 256
_H, _D = 8, 64
_HD = _H * _D
_K = 32
_TR = 256
_PR = 512
_NT = _N // _TR
_NW = 32
_NCH = _N // 16
_BH = 4


def _sim_kernel(xr_ref, xb_ref, s_ref, p_ref):
    rt = pl.program_id(1)
    sim = jax.lax.dot_general(xr_ref[0], xb_ref[0], (((1,), (1,)), ((), ())),
                              preferred_element_type=jnp.float32)
    rows = rt * _TR + jax.lax.broadcasted_iota(jnp.int32, (_TR, _N), 0)
    cols = jax.lax.broadcasted_iota(jnp.int32, (_TR, _N), 1)
    sim = sim - jnp.where(rows == cols, 1e9, 0.0).astype(jnp.float32)
    s_ref[...] = sim
    p_ref[0] = jnp.max(sim.reshape(_TR // 16, 16, _N), axis=1)


def _sim(x):
    nb = x.shape[0]
    return pl.pallas_call(
        _sim_kernel,
        grid=(nb, _NT),
        in_specs=[pl.BlockSpec((1, _TR, _C), lambda b, r: (b, r, 0)),
                  pl.BlockSpec((1, _N, _C), lambda b, r: (b, 0, 0))],
        out_specs=[pl.BlockSpec((_TR, _N), lambda b, r: (b * _NT + r, 0)),
                   pl.BlockSpec((1, _TR // 16, _N), lambda b, r: (b, r, 0))],
        out_shape=[jax.ShapeDtypeStruct((nb * _N, _N), jnp.float32),
                   jax.ShapeDtypeStruct((nb, _NCH, _N), jnp.float32)],
    )(x, x)


def _sort16(v):
    s, _ = plsc.sort_key_val(v, v, descending=True)
    return s


def _topk_thresholds(s_flat, p):
    nrow = s_flat.shape[0]
    rpw = nrow // _NW
    wpb = _N // rpw
    mesh = plsc.VectorSubcoreMesh(core_axis_name="c", subcore_axis_name="s")

    @functools.partial(
        pl.kernel,
        out_type=jax.ShapeDtypeStruct((nrow,), jnp.float32),
        mesh=mesh,
        compiler_params=pltpu.CompilerParams(needs_layout_passes=False),
        scratch_types=[
            pltpu.VMEM((_N,), jnp.float32),
            pltpu.VMEM((_N,), jnp.float32),
            pltpu.VMEM((_N,), jnp.float32),
            pltpu.VMEM((_N,), jnp.float32),
            pltpu.VMEM((_NCH, rpw), jnp.float32),
            pltpu.VMEM((_N + 16,), jnp.float32),
            pltpu.VMEM((_N + 16,), jnp.float32),
            pltpu.VMEM((_NCH + 16,), jnp.int32),
            pltpu.VMEM((_NCH + 16,), jnp.int32),
            pltpu.VMEM((rpw,), jnp.float32),
            pltpu.SemaphoreType.DMA,
            pltpu.SemaphoreType.DMA,
            pltpu.SemaphoreType.DMA,
            pltpu.SemaphoreType.DMA,
        ],
    )
    def k(s_hbm, p_hbm, t_hbm, ra0, rb0, ra1, rb1, mslab,
          cand_a, cand_b, cidx_a, cidx_b, tbuf, s0, s1, s2, s3):
        wid = lax.axis_index("s") * 2 + lax.axis_index("c")
        base = wid * rpw
        bi = wid // wpb
        c0 = (wid % wpb) * rpw
        neg = jnp.full((16,), -jnp.inf, dtype=jnp.float32)
        zeros_i = jnp.zeros((16,), jnp.int32)
        lanes = lax.iota(jnp.int32, 16)

        pltpu.sync_copy(p_hbm.at[bi, :, pl.ds(c0, rpw)], mslab)

        for i in range((_NCH + 16) // 16):
            cidx_a[pl.ds(i * 16, 16)] = zeros_i
            cidx_b[pl.ds(i * 16, 16)] = zeros_i

        def bound_t0(rl_vec):
            m1a, m2a = neg, neg
            m1b, m2b = neg, neg
            for u in range(8):
                v = plsc.load_gather(mslab, [lanes + u * 16, rl_vec])
                if u % 2 == 0:
                    m2a = jnp.maximum(m2a, jnp.minimum(m1a, v))
                    m1a = jnp.maximum(m1a, v)
                else:
                    m2b = jnp.maximum(m2b, jnp.minimum(m1b, v))
                    m1b = jnp.maximum(m1b, v)
            m2 = jnp.maximum(jnp.minimum(m1a, m1b),
                             jnp.maximum(m2a, m2b))
            return jnp.min(m2)

        def chunk_sel(t0, rl_vec, cidx):
            offc = jnp.int32(0)
            for u in range(8):
                mv = plsc.load_gather(mslab, [lanes + u * 16, rl_vec])
                cm = mv >= t0
                plsc.store_compressed(cidx.at[pl.ds(offc, 16)],
                                      lanes + u * 16, mask=cm)
                offc = offc + plsc.all_reduce_population_count(cm)[0]
            cidx[pl.ds(offc, 16)] = zeros_i
            return offc

        def process2(rowx, rowy, rl, tvec):
            rlx_vec = jnp.full((16,), rl, jnp.int32)
            rly_vec = jnp.full((16,), rl + 1, jnp.int32)
            t0x = bound_t0(rlx_vec)
            t0y = bound_t0(rly_vec)
            offcx = chunk_sel(t0x, rlx_vec, cidx_a)
            offcy = chunk_sel(t0y, rly_vec, cidx_b)

            def p2(jj, offs):
                ox, oy = offs
                cvx = cidx_a[pl.ds(jj * 16, 16)]
                cvy = cidx_b[pl.ds(jj * 16, 16)]
                for l in range(16):
                    gx = cvx[l]
                    vx = rowx[pl.ds(gx * 16, 16)]
                    mx = jnp.logical_and(vx >= t0x, jj * 16 + l < offcx)
                    plsc.store_compressed(cand_a.at[pl.ds(ox, 16)], vx,
                                          mask=mx)
                    ox = ox + plsc.all_reduce_population_count(mx)[0]
                    gy = cvy[l]
                    vy = rowy[pl.ds(gy * 16, 16)]
                    my = jnp.logical_and(vy >= t0y, jj * 16 + l < offcy)
                    plsc.store_compressed(cand_b.at[pl.ds(oy, 16)], vy,
                                          mask=my)
                    oy = oy + plsc.all_reduce_population_count(my)[0]
                return ox, oy

            ncvm = jnp.maximum(offcx, offcy)
            offx, offy = lax.fori_loop(0, (ncvm + 15) // 16, p2,
                                       (jnp.int32(0), jnp.int32(0)))
            cand_a[pl.ds(offx, 16)] = neg
            cand_b[pl.ds(offy, 16)] = neg
            nvx = (offx + 15) // 16
            nvy = (offy + 15) // 16
            nvm = jnp.maximum(nvx, nvy)

            def pad_a(i, c):
                cand_a[pl.ds(i * 16, 16)] = neg
                return c

            def pad_b(i, c):
                cand_b[pl.ds(i * 16, 16)] = neg
                return c

            lax.fori_loop(nvx, nvm, pad_a, 0)
            lax.fori_loop(nvy, nvm, pad_b, 0)

            def p3(i, st):
                ax, bx, ay, by = st
                vsx = _sort16(cand_a[pl.ds(i * 16, 16)])
                x1x = jnp.maximum(bx, lax.rev(vsx, (0,)))
                x1rx = lax.rev(_sort16(x1x), (0,))
                hix = jnp.maximum(ax, x1rx)
                lox = jnp.minimum(ax, x1rx)
                vsy = _sort16(cand_b[pl.ds(i * 16, 16)])
                x1y = jnp.maximum(by, lax.rev(vsy, (0,)))
                x1ry = lax.rev(_sort16(x1y), (0,))
                hiy = jnp.maximum(ay, x1ry)
                loy = jnp.minimum(ay, x1ry)
                return (_sort16(hix), _sort16(lox),
                        _sort16(hiy), _sort16(loy))

            _, bx, _, by = lax.fori_loop(0, nvm, p3, (neg, neg, neg, neg))
            tx = jnp.min(bx)
            ty = jnp.min(by)

            tvec = jnp.where(lanes == rl % 16, tx, tvec)
            tvec = jnp.where(lanes == (rl + 1) % 16, ty, tvec)

            @pl.when((rl + 1) % 16 == 15)
            def _():
                tbuf[pl.ds(rl - 14, 16)] = tvec

            return tvec

        pltpu.async_copy(s_hbm.at[base], ra0, s0)
        pltpu.async_copy(s_hbm.at[base + 1], rb0, s1)
        pltpu.async_copy(s_hbm.at[base + 2], ra1, s2)
        pltpu.async_copy(s_hbm.at[base + 3], rb1, s3)

        def quad(j, tvec):
            r0 = base + 4 * j
            pltpu.make_async_copy(s_hbm.at[r0], ra0, s0).wait()
            pltpu.make_async_copy(s_hbm.at[r0 + 1], rb0, s1).wait()
            tvec = process2(ra0, rb0, 4 * j, tvec)

            @pl.when(j < rpw // 4 - 1)
            def _():
                pltpu.async_copy(s_hbm.at[r0 + 4], ra0, s0)
                pltpu.async_copy(s_hbm.at[r0 + 5], rb0, s1)

            pltpu.make_async_copy(s_hbm.at[r0 + 2], ra1, s2).wait()
            pltpu.make_async_copy(s_hbm.at[r0 + 3], rb1, s3).wait()
            tvec = process2(ra1, rb1, 4 * j + 2, tvec)

            @pl.when(j < rpw // 4 - 1)
            def _():
                pltpu.async_copy(s_hbm.at[r0 + 6], ra1, s2)
                pltpu.async_copy(s_hbm.at[r0 + 7], rb1, s3)

            return tvec

        lax.fori_loop(0, rpw // 4, quad, jnp.zeros((16,), jnp.float32))
        pltpu.sync_copy(tbuf, t_hbm.at[pl.ds(base, rpw)])

    return k(s_flat, p).reshape(nrow, 1)


def _proj_kernel(x_ref, wq_ref, wk_ref, wv_ref, ws_ref,
                 q_ref, k_ref, v_ref, s_ref):
    x = x_ref[...]
    q_ref[...] = jnp.dot(x, wq_ref[...], preferred_element_type=jnp.float32)
    k_ref[...] = jnp.dot(x, wk_ref[...], preferred_element_type=jnp.float32)
    v_ref[...] = jnp.dot(x, wv_ref[...], preferred_element_type=jnp.float32)
    s_ref[...] = jnp.dot(x, ws_ref[...], preferred_element_type=jnp.float32)


def _project(xf, Wq, Wk, Wv, Wskip):
    nrow = xf.shape[0]
    wspec = pl.BlockSpec((_C, _HD), lambda i: (0, 0))
    rspec = pl.BlockSpec((_PR, _HD), lambda i: (i, 0))
    return pl.pallas_call(
        _proj_kernel,
        grid=(nrow // _PR,),
        in_specs=[pl.BlockSpec((_PR, _C), lambda i: (i, 0)),
                  wspec, wspec, wspec, wspec],
        out_specs=[rspec, rspec, rspec, rspec],
        out_shape=[jax.ShapeDtypeStruct((nrow, _HD), jnp.float32),
                   jax.ShapeDtypeStruct((nrow, _HD), jnp.float32),
                   jax.ShapeDtypeStruct((nrow, _HD), jnp.float32),
                   jax.ShapeDtypeStruct((nrow, _HD), jnp.float32)],
    )(xf, Wq, Wk, Wv, Wskip)


def _attn_kernel(s_ref, t_ref, q_ref, k_ref, v_ref, skip_ref, o_ref):
    maskf = (s_ref[...] >= t_ref[...]).astype(jnp.float32)
    skip = skip_ref[0]
    for h in range(_H):
        sl = slice(h * _D, (h + 1) * _D)
        qh = q_ref[0][:, sl]
        kh = k_ref[0][:, sl]
        vh = v_ref[0][:, sl]
        logits = jax.lax.dot_general(qh, kh, (((1,), (1,)), ((), ())),
                                     preferred_element_type=jnp.float32)
        m = jnp.max(logits, axis=1, keepdims=True)
        e = jnp.exp(logits - m) * maskf
        ssum = jnp.sum(e, axis=1, keepdims=True)
        oh = jnp.dot(e, vh, preferred_element_type=jnp.float32)
        o_ref[0, :, sl] = oh * (1.0 / (ssum + 1e-16)) + skip[:, sl]


def _attention(s_flat, thr, q, k, v, skip):
    nb = q.shape[0]
    row3 = pl.BlockSpec((1, _TR, _HD), lambda b, r: (b, r, 0))
    full3 = pl.BlockSpec((1, _N, _HD), lambda b, r: (b, 0, 0))
    return pl.pallas_call(
        _attn_kernel,
        grid=(nb, _NT),
        in_specs=[pl.BlockSpec((_TR, _N), lambda b, r: (b * _NT + r, 0)),
                  pl.BlockSpec((_TR, 1), lambda b, r: (b * _NT + r, 0)),
                  row3, full3, full3, row3],
        out_specs=row3,
        out_shape=jax.ShapeDtypeStruct((nb, _N, _HD), jnp.float32),
    )(s_flat, thr, q, k, v, skip)


def _half(xh, Wq, Wk, Wv, Wskip):
    xf = xh.reshape(_BH * _N, _C)
    s_flat, p = _sim(xh)
    thr = _topk_thresholds(s_flat, p)
    q, k, v, skip = _project(xf, Wq, Wk, Wv, Wskip)
    q = q.reshape(_BH, _N, _HD)
    k = k.reshape(_BH, _N, _HD)
    v = v.reshape(_BH, _N, _HD)
    skip = skip.reshape(_BH, _N, _HD)
    return _attention(s_flat, thr, q, k, v, skip)


def kernel(x, Wq, Wk, Wv, Wskip):
    Wq = Wq * jnp.float32(1.0 / (_D ** 0.5))
    return _half(x, Wq, Wk, Wv, Wskip)


# speedup vs baseline: 1.0996x; 1.0996x over previous
import functools

import jax
import jax.numpy as jnp
from jax import lax
from jax.experimental import pallas as pl
from jax.experimental.pallas import tpu as pltpu
from jax.experimental.pallas import tpu_sc as plsc

_B, _N, _C = 4, 2048, 256
_H, _D = 8, 64
_HD = _H * _D
_K = 32
_TR = 256
_PR = 512
_NT = _N // _TR
_NW = 32
_NCH = _N // 16
_BH = 4


def _sim_kernel(xr_ref, xb_ref, s_ref, p_ref):
    rt = pl.program_id(1)
    sim = jax.lax.dot_general(xr_ref[0], xb_ref[0], (((1,), (1,)), ((), ())),
                              preferred_element_type=jnp.float32)
    rows = rt * _TR + jax.lax.broadcasted_iota(jnp.int32, (_TR, _N), 0)
    cols = jax.lax.broadcasted_iota(jnp.int32, (_TR, _N), 1)
    sim = sim - jnp.where(rows == cols, 1e9, 0.0).astype(jnp.float32)
    s_ref[...] = sim
    p_ref[0] = jnp.max(sim.reshape(_TR // 16, 16, _N), axis=1)


def _sim(x):
    nb = x.shape[0]
    return pl.pallas_call(
        _sim_kernel,
        grid=(nb, _NT),
        in_specs=[pl.BlockSpec((1, _TR, _C), lambda b, r: (b, r, 0)),
                  pl.BlockSpec((1, _N, _C), lambda b, r: (b, 0, 0))],
        out_specs=[pl.BlockSpec((_TR, _N), lambda b, r: (b * _NT + r, 0)),
                   pl.BlockSpec((1, _TR // 16, _N), lambda b, r: (b, r, 0))],
        out_shape=[jax.ShapeDtypeStruct((nb * _N, _N), jnp.float32),
                   jax.ShapeDtypeStruct((nb, _NCH, _N), jnp.float32)],
    )(x, x)


def _sort16(v):
    s, _ = plsc.sort_key_val(v, v, descending=True)
    return s


def _topk_thresholds(s_flat, p):
    nrow = s_flat.shape[0]
    rpw = nrow // _NW
    wpb = _N // rpw
    mesh = plsc.VectorSubcoreMesh(core_axis_name="c", subcore_axis_name="s")

    @functools.partial(
        pl.kernel,
        out_type=jax.ShapeDtypeStruct((nrow,), jnp.float32),
        mesh=mesh,
        compiler_params=pltpu.CompilerParams(needs_layout_passes=False),
        scratch_types=[
            pltpu.VMEM((_N,), jnp.float32),
            pltpu.VMEM((_N,), jnp.float32),
            pltpu.VMEM((_NCH, rpw), jnp.float32),
            pltpu.VMEM((_N + 16,), jnp.float32),
            pltpu.VMEM((_NCH + 16,), jnp.int32),
            pltpu.VMEM((rpw,), jnp.float32),
            pltpu.SemaphoreType.DMA,
            pltpu.SemaphoreType.DMA,
        ],
    )
    def k(s_hbm, p_hbm, t_hbm, rowa, rowb, mslab, cand, cidx, tbuf,
          sema, semb):
        wid = lax.axis_index("s") * 2 + lax.axis_index("c")
        base = wid * rpw
        bi = wid // wpb
        c0 = (wid % wpb) * rpw
        neg = jnp.full((16,), -jnp.inf, dtype=jnp.float32)
        zeros_i = jnp.zeros((16,), jnp.int32)
        lanes = lax.iota(jnp.int32, 16)

        pltpu.sync_copy(p_hbm.at[bi, :, pl.ds(c0, rpw)], mslab)

        def process(row_ref, rl, tvec):
            rl_vec = jnp.full((16,), rl, jnp.int32)

            m1a, m2a = neg, neg
            m1b, m2b = neg, neg
            for u in range(8):
                v = plsc.load_gather(mslab, [lanes + u * 16, rl_vec])
                if u % 2 == 0:
                    m2a = jnp.maximum(m2a, jnp.minimum(m1a, v))
                    m1a = jnp.maximum(m1a, v)
                else:
                    m2b = jnp.maximum(m2b, jnp.minimum(m1b, v))
                    m1b = jnp.maximum(m1b, v)
            m2 = jnp.maximum(jnp.minimum(m1a, m1b),
                             jnp.maximum(m2a, m2b))
            t0 = jnp.min(m2)

            offc = jnp.int32(0)
            for u in range(8):
                mv = plsc.load_gather(mslab, [lanes + u * 16, rl_vec])
                cm = mv >= t0
                plsc.store_compressed(cidx.at[pl.ds(offc, 16)],
                                      lanes + u * 16, mask=cm)
                offc = offc + plsc.all_reduce_population_count(cm)[0]
            cidx[pl.ds(offc, 16)] = zeros_i

            def p2(jj, off):
                cv = cidx[pl.ds(jj * 16, 16)]
                for l in range(16):
                    g = cv[l]
                    v = row_ref[pl.ds(g * 16, 16)]
                    msk = jnp.logical_and(v >= t0, jj * 16 + l < offc)
                    plsc.store_compressed(cand.at[pl.ds(off, 16)], v,
                                          mask=msk)
                    off = off + plsc.all_reduce_population_count(msk)[0]
                return off

            off = lax.fori_loop(0, (offc + 15) // 16, p2, jnp.int32(0))
            cand[pl.ds(off, 16)] = neg

            def p3(i, ab):
                a, b = ab
                vs = _sort16(cand[pl.ds(i * 16, 16)])
                x1 = jnp.maximum(b, lax.rev(vs, (0,)))
                x1r = lax.rev(_sort16(x1), (0,))
                hi = jnp.maximum(a, x1r)
                lo = jnp.minimum(a, x1r)
                return _sort16(hi), _sort16(lo)

            _, b = lax.fori_loop(0, (off + 15) // 16, p3, (neg, neg))
            t = jnp.min(b)

            tvec = jnp.where(lanes == rl % 16, t, tvec)

            @pl.when(rl % 16 == 15)
            def _():
                tbuf[pl.ds(rl - 15, 16)] = tvec

            return tvec

        pltpu.async_copy(s_hbm.at[base], rowa, sema)

        def row_pair(j, tvec):
            r0 = base + 2 * j
            pltpu.async_copy(s_hbm.at[r0 + 1], rowb, semb)
            pltpu.make_async_copy(s_hbm.at[r0], rowa, sema).wait()
            tvec = process(rowa, 2 * j, tvec)

            @pl.when(j < rpw // 2 - 1)
            def _():
                pltpu.async_copy(s_hbm.at[r0 + 2], rowa, sema)

            pltpu.make_async_copy(s_hbm.at[r0 + 1], rowb, semb).wait()
            tvec = process(rowb, 2 * j + 1, tvec)
            return tvec

        lax.fori_loop(0, rpw // 2, row_pair, jnp.zeros((16,), jnp.float32))
        pltpu.sync_copy(tbuf, t_hbm.at[pl.ds(base, rpw)])

    return k(s_flat, p).reshape(nrow, 1)


def _proj_kernel(x_ref, wq_ref, wk_ref, wv_ref, ws_ref,
                 q_ref, k_ref, v_ref, s_ref):
    x = x_ref[...]
    q_ref[...] = jnp.dot(x, wq_ref[...], preferred_element_type=jnp.float32)
    k_ref[...] = jnp.dot(x, wk_ref[...], preferred_element_type=jnp.float32)
    v_ref[...] = jnp.dot(x, wv_ref[...], preferred_element_type=jnp.float32)
    s_ref[...] = jnp.dot(x, ws_ref[...], preferred_element_type=jnp.float32)


def _project(xf, Wq, Wk, Wv, Wskip):
    nrow = xf.shape[0]
    wspec = pl.BlockSpec((_C, _HD), lambda i: (0, 0))
    rspec = pl.BlockSpec((_PR, _HD), lambda i: (i, 0))
    return pl.pallas_call(
        _proj_kernel,
        grid=(nrow // _PR,),
        in_specs=[pl.BlockSpec((_PR, _C), lambda i: (i, 0)),
                  wspec, wspec, wspec, wspec],
        out_specs=[rspec, rspec, rspec, rspec],
        out_shape=[jax.ShapeDtypeStruct((nrow, _HD), jnp.float32),
                   jax.ShapeDtypeStruct((nrow, _HD), jnp.float32),
                   jax.ShapeDtypeStruct((nrow, _HD), jnp.float32),
                   jax.ShapeDtypeStruct((nrow, _HD), jnp.float32)],
    )(xf, Wq, Wk, Wv, Wskip)


def _attn_kernel(s_ref, t_ref, q_ref, k_ref, v_ref, skip_ref, o_ref):
    maskf = (s_ref[...] >= t_ref[...]).astype(jnp.float32)
    skip = skip_ref[0]
    for h in range(_H):
        sl = slice(h * _D, (h + 1) * _D)
        qh = q_ref[0][:, sl]
        kh = k_ref[0][:, sl]
        vh = v_ref[0][:, sl]
        logits = jax.lax.dot_general(qh, kh, (((1,), (1,)), ((), ())),
                                     preferred_element_type=jnp.float32)
        m = jnp.max(logits, axis=1, keepdims=True)
        e = jnp.exp(logits - m) * maskf
        ssum = jnp.sum(e, axis=1, keepdims=True)
        oh = jnp.dot(e, vh, preferred_element_type=jnp.float32)
        o_ref[0, :, sl] = oh * (1.0 / (ssum + 1e-16)) + skip[:, sl]


def _attention(s_flat, thr, q, k, v, skip):
    nb = q.shape[0]
    row3 = pl.BlockSpec((1, _TR, _HD), lambda b, r: (b, r, 0))
    full3 = pl.BlockSpec((1, _N, _HD), lambda b, r: (b, 0, 0))
    return pl.pallas_call(
        _attn_kernel,
        grid=(nb, _NT),
        in_specs=[pl.BlockSpec((_TR, _N), lambda b, r: (b * _NT + r, 0)),
                  pl.BlockSpec((_TR, 1), lambda b, r: (b * _NT + r, 0)),
                  row3, full3, full3, row3],
        out_specs=row3,
        out_shape=jax.ShapeDtypeStruct((nb, _N, _HD), jnp.float32),
    )(s_flat, thr, q, k, v, skip)


def _half(xh, Wq, Wk, Wv, Wskip):
    xf = xh.reshape(_BH * _N, _C)
    s_flat, p = _sim(xh)
    thr = _topk_thresholds(s_flat, p)
    q, k, v, skip = _project(xf, Wq, Wk, Wv, Wskip)
    q = q.reshape(_BH, _N, _HD)
    k = k.reshape(_BH, _N, _HD)
    v = v.reshape(_BH, _N, _HD)
    skip = skip.reshape(_BH, _N, _HD)
    return _attention(s_flat, thr, q, k, v, skip)


def kernel(x, Wq, Wk, Wv, Wskip):
    Wq = Wq * jnp.float32(1.0 / (_D ** 0.5))
    return _half(x, Wq, Wk, Wv, Wskip)


# speedup vs baseline: 1.1060x; 1.0058x over previous
import functools

import jax
import jax.numpy as jnp
from jax import lax
from jax.experimental import pallas as pl
from jax.experimental.pallas import tpu as pltpu
from jax.experimental.pallas import tpu_sc as plsc

_B, _N, _C = 4, 2048, 256
_H, _D = 8, 64
_HD = _H * _D
_K = 32
_TR = 256
_PR = 512
_NT = _N // _TR
_NW = 32
_NCH = _N // 16
_BH = 4


def _sim_kernel(xr_ref, xb_ref, s_ref, p_ref):
    rt = pl.program_id(1)
    sim = jax.lax.dot_general(xr_ref[0], xb_ref[0], (((1,), (1,)), ((), ())),
                              preferred_element_type=jnp.float32)
    rows = rt * _TR + jax.lax.broadcasted_iota(jnp.int32, (_TR, _N), 0)
    cols = jax.lax.broadcasted_iota(jnp.int32, (_TR, _N), 1)
    sim = sim - jnp.where(rows == cols, 1e9, 0.0).astype(jnp.float32)
    s_ref[...] = sim
    p_ref[0] = jnp.max(sim.reshape(_TR // 16, 16, _N), axis=1)


def _sim(x):
    nb = x.shape[0]
    return pl.pallas_call(
        _sim_kernel,
        grid=(nb, _NT),
        in_specs=[pl.BlockSpec((1, _TR, _C), lambda b, r: (b, r, 0)),
                  pl.BlockSpec((1, _N, _C), lambda b, r: (b, 0, 0))],
        out_specs=[pl.BlockSpec((_TR, _N), lambda b, r: (b * _NT + r, 0)),
                   pl.BlockSpec((1, _TR // 16, _N), lambda b, r: (b, r, 0))],
        out_shape=[jax.ShapeDtypeStruct((nb * _N, _N), jnp.float32),
                   jax.ShapeDtypeStruct((nb, _NCH, _N), jnp.float32)],
    )(x, x)


def _sort16(v):
    s, _ = plsc.sort_key_val(v, v, descending=True)
    return s


def _topk_thresholds(s_flat, p):
    nrow = s_flat.shape[0]
    rpw = nrow // _NW
    wpb = _N // rpw
    mesh = plsc.VectorSubcoreMesh(core_axis_name="c", subcore_axis_name="s")

    @functools.partial(
        pl.kernel,
        out_type=jax.ShapeDtypeStruct((nrow,), jnp.float32),
        mesh=mesh,
        compiler_params=pltpu.CompilerParams(needs_layout_passes=False),
        scratch_types=[
            pltpu.VMEM((_N,), jnp.float32),
            pltpu.VMEM((_N,), jnp.float32),
            pltpu.VMEM((_NCH, rpw), jnp.float32),
            pltpu.VMEM((_N + 16,), jnp.float32),
            pltpu.VMEM((_NCH + 16,), jnp.int32),
            pltpu.VMEM((rpw,), jnp.float32),
            pltpu.SemaphoreType.DMA,
            pltpu.SemaphoreType.DMA,
        ],
    )
    def k(s_hbm, p_hbm, t_hbm, rowa, rowb, mslab, cand, cidx, tbuf,
          sema, semb):
        wid = lax.axis_index("s") * 2 + lax.axis_index("c")
        base = wid * rpw
        bi = wid // wpb
        c0 = (wid % wpb) * rpw
        neg = jnp.full((16,), -jnp.inf, dtype=jnp.float32)
        zeros_i = jnp.zeros((16,), jnp.int32)
        lanes = lax.iota(jnp.int32, 16)

        pltpu.sync_copy(p_hbm.at[bi, :, pl.ds(c0, rpw)], mslab)

        def process(row_ref, rl, tvec):
            rl_vec = jnp.full((16,), rl, jnp.int32)

            m1a, m2a = neg, neg
            m1b, m2b = neg, neg
            for u in range(8):
                v = plsc.load_gather(mslab, [lanes + u * 16, rl_vec])
                if u % 2 == 0:
                    m2a = jnp.maximum(m2a, jnp.minimum(m1a, v))
                    m1a = jnp.maximum(m1a, v)
                else:
                    m2b = jnp.maximum(m2b, jnp.minimum(m1b, v))
                    m1b = jnp.maximum(m1b, v)
            m2 = jnp.maximum(jnp.minimum(m1a, m1b),
                             jnp.maximum(m2a, m2b))
            t0 = jnp.min(m2)

            offc = jnp.int32(0)
            for u in range(8):
                mv = plsc.load_gather(mslab, [lanes + u * 16, rl_vec])
                cm = mv >= t0
                plsc.store_compressed(cidx.at[pl.ds(offc, 16)],
                                      lanes + u * 16, mask=cm)
                offc = offc + plsc.all_reduce_population_count(cm)[0]
            cidx[pl.ds(offc, 16)] = zeros_i

            def p2(jj, off):
                cv = cidx[pl.ds(jj * 16, 16)]
                for l in range(16):
                    g = cv[l]
                    v = row_ref[pl.ds(g * 16, 16)]
                    msk = jnp.logical_and(v >= t0, jj * 16 + l < offc)
                    plsc.store_compressed(cand.at[pl.ds(off, 16)], v,
                                          mask=msk)
                    off = off + plsc.all_reduce_population_count(msk)[0]
                return off

            off = lax.fori_loop(0, (offc + 15) // 16, p2, jnp.int32(0))
            cand[pl.ds(off, 16)] = neg

            def p3(i, ab):
                a, b = ab
                vs = _sort16(cand[pl.ds(i * 16, 16)])
                x1 = jnp.maximum(b, lax.rev(vs, (0,)))
                x1r = lax.rev(_sort16(x1), (0,))
                hi = jnp.maximum(a, x1r)
                lo = jnp.minimum(a, x1r)
                return _sort16(hi), _sort16(lo)

            _, b = lax.fori_loop(0, (off + 15) // 16, p3, (neg, neg))
            t = jnp.min(b)

            tvec = jnp.where(lanes == rl % 16, t, tvec)

            @pl.when(rl % 16 == 15)
            def _():
                tbuf[pl.ds(rl - 15, 16)] = tvec

            return tvec

        pltpu.async_copy(s_hbm.at[base], rowa, sema)

        def row_pair(j, tvec):
            r0 = base + 2 * j
            pltpu.async_copy(s_hbm.at[r0 + 1], rowb, semb)
            pltpu.make_async_copy(s_hbm.at[r0], rowa, sema).wait()
            tvec = process(rowa, 2 * j, tvec)

            @pl.when(j < rpw // 2 - 1)
            def _():
                pltpu.async_copy(s_hbm.at[r0 + 2], rowa, sema)

            pltpu.make_async_copy(s_hbm.at[r0 + 1], rowb, semb).wait()
            tvec = process(rowb, 2 * j + 1, tvec)
            return tvec

        lax.fori_loop(0, rpw // 2, row_pair, jnp.zeros((16,), jnp.float32))
        pltpu.sync_copy(tbuf, t_hbm.at[pl.ds(base, rpw)])

    return k(s_flat, p).reshape(nrow, 1)


def _proj_kernel(x_ref, wq_ref, wk_ref, wv_ref, ws_ref,
                 q_ref, k_ref, v_ref, s_ref):
    x = x_ref[...]
    q_ref[...] = jnp.dot(x, wq_ref[...], preferred_element_type=jnp.float32)
    k_ref[...] = jnp.dot(x, wk_ref[...], preferred_element_type=jnp.float32)
    v_ref[...] = jnp.dot(x, wv_ref[...], preferred_element_type=jnp.float32)
    s_ref[...] = jnp.dot(x, ws_ref[...], preferred_element_type=jnp.float32)


def _project(xf, Wq, Wk, Wv, Wskip):
    nrow = xf.shape[0]
    wspec = pl.BlockSpec((_C, _HD), lambda i: (0, 0))
    rspec = pl.BlockSpec((_PR, _HD), lambda i: (i, 0))
    return pl.pallas_call(
        _proj_kernel,
        grid=(nrow // _PR,),
        in_specs=[pl.BlockSpec((_PR, _C), lambda i: (i, 0)),
                  wspec, wspec, wspec, wspec],
        out_specs=[rspec, rspec, rspec, rspec],
        out_shape=[jax.ShapeDtypeStruct((nrow, _HD), jnp.float32),
                   jax.ShapeDtypeStruct((nrow, _HD), jnp.float32),
                   jax.ShapeDtypeStruct((nrow, _HD), jnp.float32),
                   jax.ShapeDtypeStruct((nrow, _HD), jnp.float32)],
    )(xf, Wq, Wk, Wv, Wskip)


def _attn_kernel(s_ref, t_ref, q_ref, k_ref, v_ref, skip_ref, o_ref):
    maskf = (s_ref[...] >= t_ref[...]).astype(jnp.float32)
    skip = skip_ref[0]
    for h in range(_H):
        sl = slice(h * _D, (h + 1) * _D)
        qh = q_ref[0][:, sl]
        kh = k_ref[0][:, sl]
        vh = v_ref[0][:, sl]
        logits = jax.lax.dot_general(qh, kh, (((1,), (1,)), ((), ())),
                                     preferred_element_type=jnp.float32)
        m = jnp.max(logits, axis=1, keepdims=True)
        e = jnp.exp2(logits - m) * maskf
        ssum = jnp.sum(e, axis=1, keepdims=True)
        oh = jnp.dot(e, vh, preferred_element_type=jnp.float32)
        o_ref[0, :, sl] = oh * (1.0 / (ssum + 1e-16)) + skip[:, sl]


def _attention(s_flat, thr, q, k, v, skip):
    nb = q.shape[0]
    row3 = pl.BlockSpec((1, _TR, _HD), lambda b, r: (b, r, 0))
    full3 = pl.BlockSpec((1, _N, _HD), lambda b, r: (b, 0, 0))
    return pl.pallas_call(
        _attn_kernel,
        grid=(nb, _NT),
        in_specs=[pl.BlockSpec((_TR, _N), lambda b, r: (b * _NT + r, 0)),
                  pl.BlockSpec((_TR, 1), lambda b, r: (b * _NT + r, 0)),
                  row3, full3, full3, row3],
        out_specs=row3,
        out_shape=jax.ShapeDtypeStruct((nb, _N, _HD), jnp.float32),
    )(s_flat, thr, q, k, v, skip)


def _half(xh, Wq, Wk, Wv, Wskip):
    xf = xh.reshape(_BH * _N, _C)
    s_flat, p = _sim(xh)
    thr = _topk_thresholds(s_flat, p)
    q, k, v, skip = _project(xf, Wq, Wk, Wv, Wskip)
    q = q.reshape(_BH, _N, _HD)
    k = k.reshape(_BH, _N, _HD)
    v = v.reshape(_BH, _N, _HD)
    skip = skip.reshape(_BH, _N, _HD)
    return _attention(s_flat, thr, q, k, v, skip)


def kernel(x, Wq, Wk, Wv, Wskip):
    Wq = Wq * jnp.float32(1.4426950408889634 / (_D ** 0.5))
    return _half(x, Wq, Wk, Wv, Wskip)
